# baseline (speedup 1.0000x reference)
import functools

import jax
import jax.numpy as jnp
from jax import lax
from jax.experimental import pallas as pl
from jax.experimental.pallas import tpu as pltpu
from jax.experimental.pallas import tpu_sc as plsc

N = 16384
D = 64
V = 8192
R = 512
VC = 4096

_NC = 2
_NS = 16
_NW = _NC * _NS
_BPW = N // _NW


def _tokens_body(coords_ref, feats_ref, W1s_ref, b1s_ref, W2s_ref, b2s_ref,
                 W1f_ref, b1f_ref, W2f_ref, b2f_ref, cb_ref, tok_ref):
    def bdot(a, b, dims):
        return jax.lax.dot_general(a.astype(jnp.bfloat16), b.astype(jnp.bfloat16),
                                   dims, preferred_element_type=jnp.float32)

    coords = coords_ref[...]
    feats = feats_ref[...]
    nt = (((1,), (0,)), ((), ()))
    h_s = jnp.maximum(bdot(coords, W1s_ref[...], nt) + b1s_ref[...][None, :], 0.0)
    e_s = bdot(h_s, W2s_ref[...], nt) + b2s_ref[...][None, :]
    h_f = jnp.maximum(bdot(feats, W1f_ref[...], nt) + b1f_ref[...][None, :], 0.0)
    e_f = bdot(h_f, W2f_ref[...], nt) + b2f_ref[...][None, :]
    x = e_s + e_f
    x2 = jnp.sum(x * x, axis=1, keepdims=True)

    def body(j, carry):
        best_d, best_i = carry
        cb = cb_ref[pl.ds(j * VC, VC), :]
        c2 = jnp.sum(cb * cb, axis=1)[None, :]
        dot = bdot(x, cb, (((1,), (1,)), ((), ())))
        d = jnp.sqrt(jnp.maximum(x2 + c2 - 2.0 * dot, 0.0))
        cmin = jnp.min(d, axis=1, keepdims=True)
        ids = lax.broadcasted_iota(jnp.int32, (R, VC), 1)
        cidx = jnp.min(jnp.where(d == cmin, ids, V), axis=1, keepdims=True) + j * VC
        upd = cmin < best_d
        cmin_b = cmin.astype(jnp.bfloat16).astype(jnp.float32)
        return (jnp.where(upd, cmin_b, best_d), jnp.where(upd, cidx, best_i))

    init = (jnp.full((R, 1), jnp.inf, jnp.float32), jnp.zeros((R, 1), jnp.int32))
    _, best_i = lax.fori_loop(0, V // VC, body, init)
    tok_ref[...] = best_i


def _tokens_call(coordinates, features, W1s, b1s, W2s, b2s, W1f, b1f, W2f, b2f,
                 codebook, interpret=False):
    grid = (N // R,)
    full = lambda shape: pl.BlockSpec(shape, lambda i: tuple(0 for _ in shape))
    out = pl.pallas_call(
        _tokens_body,
        grid=grid,
        in_specs=[
            pl.BlockSpec((R, 2), lambda i: (i, 0)),
            pl.BlockSpec((R, 10), lambda i: (i, 0)),
            full((2, 128)), full((128,)), full((128, D)), full((D,)),
            full((10, 128)), full((128,)), full((128, D)), full((D,)),
            full((V, D)),
        ],
        out_specs=pl.BlockSpec((R, 1), lambda i: (i, 0)),
        out_shape=jax.ShapeDtypeStruct((N, 1), jnp.int32),
        compiler_params=pltpu.CompilerParams(
            dimension_semantics=("parallel",)),
        interpret=interpret,
    )(coordinates, features, W1s, b1s, W2s, b2s, W1f, b1f, W2f, b2f, codebook)
    return out[:, 0]


@functools.lru_cache(maxsize=1)
def _make_gather_rows():
    mesh = plsc.VectorSubcoreMesh(core_axis_name="c", subcore_axis_name="s")

    @functools.partial(
        pl.kernel, mesh=mesh,
        out_type=jax.ShapeDtypeStruct((N, D), jnp.float32),
        scratch_types=[
            pltpu.VMEM((_BPW,), jnp.int32),
            pltpu.VMEM((_BPW, D), jnp.float32),
            pltpu.SemaphoreType.DMA,
        ],
        compiler_params=pltpu.CompilerParams(use_tc_tiling_on_sc=False),
    )
    def _gather_rows(cb_hbm, idx_hbm, out_hbm, idx_v, rows_v, sem):
        wid = lax.axis_index("s") * _NC + lax.axis_index("c")
        base = wid * _BPW
        pltpu.sync_copy(idx_hbm.at[pl.ds(base, _BPW)], idx_v)
        pltpu.async_copy(cb_hbm.at[idx_v], rows_v, sem).wait()
        pltpu.sync_copy(rows_v, out_hbm.at[pl.ds(base, _BPW)])

    return _gather_rows


def kernel(coordinates, features, W1s, b1s, W2s, b2s, W1f, b1f, W2f, b2f, codebook):
    tokens = _tokens_call(coordinates, features, W1s, b1s, W2s, b2s,
                          W1f, b1f, W2f, b2f, codebook)
    quantized = _make_gather_rows()(codebook, tokens)
    return tokens, quantized


# speedup vs baseline: 1.5877x; 1.5877x over previous
import functools

import jax
import jax.numpy as jnp
from jax import lax
from jax.experimental import pallas as pl
from jax.experimental.pallas import tpu as pltpu
from jax.experimental.pallas import tpu_sc as plsc

N = 16384
D = 64
V = 8192
R = 512
VC = 4096

_NC = 2
_NS = 16
_NW = _NC * _NS
_BPW = N // _NW


def _tokens_body(coords_ref, feats_ref, W1s_ref, b1s_ref, W2s_ref, b2s_ref,
                 W1f_ref, b1f_ref, W2f_ref, b2f_ref, cb_ref, tok_ref):
    def bdot(a, b, dims):
        return jax.lax.dot_general(a.astype(jnp.bfloat16), b.astype(jnp.bfloat16),
                                   dims, preferred_element_type=jnp.float32)

    coords = coords_ref[...]
    feats = feats_ref[...]
    nt = (((1,), (0,)), ((), ()))
    h_s = jnp.maximum(bdot(coords, W1s_ref[...], nt) + b1s_ref[...][None, :], 0.0)
    e_s = bdot(h_s, W2s_ref[...], nt) + b2s_ref[...][None, :]
    h_f = jnp.maximum(bdot(feats, W1f_ref[...], nt) + b1f_ref[...][None, :], 0.0)
    e_f = bdot(h_f, W2f_ref[...], nt) + b2f_ref[...][None, :]
    x = e_s + e_f
    x2 = jnp.sum(x * x, axis=1, keepdims=True)

    def body(j, carry):
        best_d, best_i = carry
        cb = cb_ref[pl.ds(j * VC, VC), :]
        c2 = jnp.sum(cb * cb, axis=1)[None, :]
        cbn = cb.astype(jnp.bfloat16) * jnp.bfloat16(-2.0)
        gneg = jax.lax.dot_general(x.astype(jnp.bfloat16), cbn,
                                   (((1,), (1,)), ((), ())),
                                   preferred_element_type=jnp.float32)
        d2 = (x2 + c2) + gneg
        cmin2 = jnp.min(d2, axis=1, keepdims=True)
        ids = lax.broadcasted_iota(jnp.int32, (R, VC), 1)
        cidx = jnp.min(jnp.where(d2 == cmin2, ids, V), axis=1, keepdims=True) + j * VC
        cmin = jnp.sqrt(jnp.maximum(cmin2, 0.0))
        upd = cmin < best_d
        cmin_b = cmin.astype(jnp.bfloat16).astype(jnp.float32)
        return (jnp.where(upd, cmin_b, best_d), jnp.where(upd, cidx, best_i))

    init = (jnp.full((R, 1), jnp.inf, jnp.float32), jnp.zeros((R, 1), jnp.int32))
    _, best_i = lax.fori_loop(0, V // VC, body, init)
    tok_ref[...] = best_i


def _tokens_call(coordinates, features, W1s, b1s, W2s, b2s, W1f, b1f, W2f, b2f,
                 codebook, interpret=False):
    grid = (N // R,)
    full = lambda shape: pl.BlockSpec(shape, lambda i: tuple(0 for _ in shape))
    out = pl.pallas_call(
        _tokens_body,
        grid=grid,
        in_specs=[
            pl.BlockSpec((R, 2), lambda i: (i, 0)),
            pl.BlockSpec((R, 10), lambda i: (i, 0)),
            full((2, 128)), full((128,)), full((128, D)), full((D,)),
            full((10, 128)), full((128,)), full((128, D)), full((D,)),
            full((V, D)),
        ],
        out_specs=pl.BlockSpec((R, 1), lambda i: (i, 0)),
        out_shape=jax.ShapeDtypeStruct((N, 1), jnp.int32),
        compiler_params=pltpu.CompilerParams(
            dimension_semantics=("parallel",)),
        interpret=interpret,
    )(coordinates, features, W1s, b1s, W2s, b2s, W1f, b1f, W2f, b2f, codebook)
    return out[:, 0]


@functools.lru_cache(maxsize=1)
def _make_gather_rows():
    mesh = plsc.VectorSubcoreMesh(core_axis_name="c", subcore_axis_name="s")

    @functools.partial(
        pl.kernel, mesh=mesh,
        out_type=jax.ShapeDtypeStruct((N, D), jnp.float32),
        scratch_types=[
            pltpu.VMEM((_BPW,), jnp.int32),
            pltpu.VMEM((_BPW, D), jnp.float32),
            pltpu.SemaphoreType.DMA,
        ],
        compiler_params=pltpu.CompilerParams(use_tc_tiling_on_sc=False),
    )
    def _gather_rows(cb_hbm, idx_hbm, out_hbm, idx_v, rows_v, sem):
        wid = lax.axis_index("s") * _NC + lax.axis_index("c")
        base = wid * _BPW
        pltpu.sync_copy(idx_hbm.at[pl.ds(base, _BPW)], idx_v)
        pltpu.async_copy(cb_hbm.at[idx_v], rows_v, sem).wait()
        pltpu.sync_copy(rows_v, out_hbm.at[pl.ds(base, _BPW)])

    return _gather_rows


def kernel(coordinates, features, W1s, b1s, W2s, b2s, W1f, b1f, W2f, b2f, codebook):
    tokens = _tokens_call(coordinates, features, W1s, b1s, W2s, b2s,
                          W1f, b1f, W2f, b2f, codebook)
    quantized = _make_gather_rows()(codebook, tokens)
    return tokens, quantized
